# baseline (speedup 1.0000x reference)
import functools

import jax
import jax.numpy as jnp
from jax import lax
from jax.experimental import pallas as pl
from jax.experimental.pallas import tpu as pltpu
from jax.experimental.pallas import tpu_sc as plsc

DIM = 2048
HID = 1024
NE = 8
TB = 512
TILE = 256


def _router_kernel(x_ref, gwt_ref, xb_ref, i0_ref, i1_ref, w0_ref, w1_ref,
                   r0_ref, r1_ref, counts_ref):
    i = pl.program_id(0)

    @pl.when(i == 0)
    def _():
        counts_ref[...] = jnp.zeros_like(counts_ref)

    x = x_ref[...]
    xb = x.astype(jnp.bfloat16)
    xb_ref[...] = xb
    logits = jnp.dot(xb, gwt_ref[...].astype(jnp.bfloat16),
                     preferred_element_type=jnp.float32)
    lane = lax.broadcasted_iota(jnp.int32, (TB, 128), 1)
    neg = jnp.float32(-1e30)
    logits = jnp.where(lane < NE, logits, neg)
    m0 = jnp.max(logits, axis=1, keepdims=True)
    i0 = jnp.min(jnp.where(logits == m0, lane, 127), axis=1, keepdims=True)
    logits1 = jnp.where(lane == i0, neg, logits)
    m1 = jnp.max(logits1, axis=1, keepdims=True)
    i1 = jnp.min(jnp.where(logits1 == m1, lane, 127), axis=1, keepdims=True)
    s0 = jax.nn.sigmoid(m0)
    s1 = jax.nn.sigmoid(m1)
    denom = s0 + s1 + jnp.float32(1e-9)

    oh = ((lane == i0) | (lane == i1)).astype(jnp.float32)
    row = lax.broadcasted_iota(jnp.int32, (TB, TB), 0)
    col = lax.broadcasted_iota(jnp.int32, (TB, TB), 1)
    L = (row > col).astype(jnp.float32)
    ranks = jnp.dot(L, oh, preferred_element_type=jnp.float32) + counts_ref[...]
    r0 = jnp.sum(jnp.where(lane == i0, ranks, 0.0), axis=1, keepdims=True)
    r1 = jnp.sum(jnp.where(lane == i1, ranks, 0.0), axis=1, keepdims=True)
    counts_ref[...] += jnp.sum(oh, axis=0, keepdims=True)

    i0_ref[...] = jnp.broadcast_to(i0, (TB, 128))
    i1_ref[...] = jnp.broadcast_to(i1, (TB, 128))
    w0_ref[...] = jnp.broadcast_to(s0 / denom, (TB, 128))
    w1_ref[...] = jnp.broadcast_to(s1 / denom, (TB, 128))
    r0_ref[...] = jnp.broadcast_to(r0, (TB, 128)).astype(jnp.int32)
    r1_ref[...] = jnp.broadcast_to(r1, (TB, 128)).astype(jnp.int32)


def _dest_kernel(counts_ref, i0_ref, i1_ref, r0_ref, r1_ref,
                 d0_ref, d1_ref, te_ref):
    counts = counts_ref[...]
    pc = jnp.ceil(counts * (1.0 / TILE)) * TILE
    row = lax.broadcasted_iota(jnp.int32, (128, 128), 0)
    col = lax.broadcasted_iota(jnp.int32, (128, 128), 1)
    LT = (row < col).astype(jnp.float32)
    offs = jnp.dot(pc, LT, preferred_element_type=jnp.float32)
    lane = lax.broadcasted_iota(jnp.int32, (TB, 128), 1)
    offs_b = jnp.broadcast_to(offs, (TB, 128))
    o0 = jnp.sum(jnp.where(lane == i0_ref[...], offs_b, 0.0), axis=1,
                 keepdims=True)
    o1 = jnp.sum(jnp.where(lane == i1_ref[...], offs_b, 0.0), axis=1,
                 keepdims=True)
    d0 = o0.astype(jnp.int32) + r0_ref[:, :1]
    d1 = o1.astype(jnp.int32) + r1_ref[:, :1]
    d0_ref[...] = jnp.broadcast_to(d0, (TB, 128))
    d1_ref[...] = jnp.broadcast_to(d1, (TB, 128))

    @pl.when(pl.program_id(0) == 0)
    def _():
        bounds = offs + pc
        lane1 = lax.broadcasted_iota(jnp.int32, (1, 128), 1)
        te = jnp.zeros((1, 128), jnp.float32)
        for e in range(NE):
            be = jnp.sum(jnp.where(lane1 == e, bounds, 0.0), axis=1,
                         keepdims=True)
            te += (jnp.broadcast_to(be, (1, 128)) <=
                   (lane1 * TILE).astype(jnp.float32)).astype(jnp.float32)
        te_ref[...] = jnp.minimum(te, NE - 1).astype(jnp.int32)


def _mlp_kernel(te_ref, xd_ref, wg_ref, wu_ref, wd_ref, y_ref):
    xb = xd_ref[...]
    dn = (((1,), (1,)), ((), ()))
    g = lax.dot_general(xb, wg_ref[0], dn, preferred_element_type=jnp.float32)
    u = lax.dot_general(xb, wu_ref[0], dn, preferred_element_type=jnp.float32)
    h = (g * jax.nn.sigmoid(g)) * u
    y_ref[...] = lax.dot_general(h.astype(jnp.bfloat16), wd_ref[0], dn,
                                 preferred_element_type=jnp.float32)


def _run_tc_stages(flat, gate_w):
    T = flat.shape[0]
    nb = T // TB
    gwt = jnp.zeros((DIM, 128), jnp.float32).at[:, :NE].set(gate_w.T)

    outs = pl.pallas_call(
        _router_kernel,
        grid=(nb,),
        in_specs=[
            pl.BlockSpec((TB, DIM), lambda i: (i, 0)),
            pl.BlockSpec((DIM, 128), lambda i: (0, 0)),
        ],
        out_specs=[
            pl.BlockSpec((TB, DIM), lambda i: (i, 0)),
            pl.BlockSpec((TB, 128), lambda i: (i, 0)),
            pl.BlockSpec((TB, 128), lambda i: (i, 0)),
            pl.BlockSpec((TB, 128), lambda i: (i, 0)),
            pl.BlockSpec((TB, 128), lambda i: (i, 0)),
            pl.BlockSpec((TB, 128), lambda i: (i, 0)),
            pl.BlockSpec((TB, 128), lambda i: (i, 0)),
            pl.BlockSpec((1, 128), lambda i: (0, 0)),
        ],
        out_shape=[
            jax.ShapeDtypeStruct((T, DIM), jnp.bfloat16),
            jax.ShapeDtypeStruct((T, 128), jnp.int32),
            jax.ShapeDtypeStruct((T, 128), jnp.int32),
            jax.ShapeDtypeStruct((T, 128), jnp.float32),
            jax.ShapeDtypeStruct((T, 128), jnp.float32),
            jax.ShapeDtypeStruct((T, 128), jnp.int32),
            jax.ShapeDtypeStruct((T, 128), jnp.int32),
            jax.ShapeDtypeStruct((1, 128), jnp.float32),
        ],
    )(flat, gwt)
    xb, i0a, i1a, w0a, w1a, r0a, r1a, counts = outs

    d0a, d1a, te_arr = pl.pallas_call(
        _dest_kernel,
        grid=(nb,),
        in_specs=[
            pl.BlockSpec((1, 128), lambda i: (0, 0)),
            pl.BlockSpec((TB, 128), lambda i: (i, 0)),
            pl.BlockSpec((TB, 128), lambda i: (i, 0)),
            pl.BlockSpec((TB, 128), lambda i: (i, 0)),
            pl.BlockSpec((TB, 128), lambda i: (i, 0)),
        ],
        out_specs=[
            pl.BlockSpec((TB, 128), lambda i: (i, 0)),
            pl.BlockSpec((TB, 128), lambda i: (i, 0)),
            pl.BlockSpec((1, 128), lambda i: (0, 0)),
        ],
        out_shape=[
            jax.ShapeDtypeStruct((T, 128), jnp.int32),
            jax.ShapeDtypeStruct((T, 128), jnp.int32),
            jax.ShapeDtypeStruct((1, 128), jnp.int32),
        ],
    )(counts, i0a, i1a, r0a, r1a)
    return xb, w0a, w1a, d0a, d1a, te_arr


def _run_mlp(xd, te, Wg_b, Wu_b, Wd_b):
    S = xd.shape[0]
    ntiles = S // TILE
    grid_spec = pltpu.PrefetchScalarGridSpec(
        num_scalar_prefetch=1,
        grid=(ntiles,),
        in_specs=[
            pl.BlockSpec((TILE, DIM), lambda i, te_r: (i, 0)),
            pl.BlockSpec((1, HID, DIM), lambda i, te_r: (te_r[i], 0, 0)),
            pl.BlockSpec((1, HID, DIM), lambda i, te_r: (te_r[i], 0, 0)),
            pl.BlockSpec((1, DIM, HID), lambda i, te_r: (te_r[i], 0, 0)),
        ],
        out_specs=pl.BlockSpec((TILE, DIM), lambda i, te_r: (i, 0)),
    )
    return pl.pallas_call(
        _mlp_kernel,
        grid_spec=grid_spec,
        out_shape=jax.ShapeDtypeStruct((S, DIM), jnp.float32),
    )(te, xd, Wg_b, Wu_b, Wd_b)


def _sc_dispatch(xbi, d0, d1, S):
    T = xbi.shape[0]
    NW = 32
    per_w = T // NW
    ROWS = 64
    ngroups = per_w // ROWS
    mesh = plsc.VectorSubcoreMesh(core_axis_name="c", subcore_axis_name="s")

    @functools.partial(
        pl.kernel, mesh=mesh,
        out_type=jax.ShapeDtypeStruct((S, DIM // 2), jnp.int32),
        scratch_types=[
            pltpu.VMEM((ROWS, DIM // 2), jnp.int32),
            pltpu.VMEM((ROWS,), jnp.int32),
            pltpu.VMEM((ROWS,), jnp.int32),
            pltpu.SemaphoreType.DMA,
            pltpu.SemaphoreType.DMA,
        ],
    )
    def k(xb_hbm, d0_hbm, d1_hbm, xd_hbm, rows_v, idx0_v, idx1_v, sem0, sem1):
        wid = lax.axis_index("s") * 2 + lax.axis_index("c")
        base = wid * per_w
        for g in range(ngroups):
            off = base + g * ROWS
            pltpu.sync_copy(xb_hbm.at[pl.ds(off, ROWS)], rows_v)
            pltpu.sync_copy(d0_hbm.at[pl.ds(off, ROWS)], idx0_v)
            pltpu.sync_copy(d1_hbm.at[pl.ds(off, ROWS)], idx1_v)
            a = pltpu.async_copy(rows_v, xd_hbm.at[idx0_v], sem0)
            b = pltpu.async_copy(rows_v, xd_hbm.at[idx1_v], sem1)
            a.wait()
            b.wait()

    return k(xbi, d0, d1)


def _sc_combine(y, d0, d1, w0a, w1a):
    T = d0.shape[0]
    NW = 32
    per_w = T // NW
    G = 16
    ngroups = per_w // G
    mesh = plsc.VectorSubcoreMesh(core_axis_name="c", subcore_axis_name="s")

    @functools.partial(
        pl.kernel, mesh=mesh,
        out_type=jax.ShapeDtypeStruct((T, DIM), jnp.float32),
        scratch_types=[
            pltpu.VMEM((G, DIM), jnp.float32),
            pltpu.VMEM((G, DIM), jnp.float32),
            pltpu.VMEM((G, DIM), jnp.float32),
            pltpu.VMEM((G,), jnp.int32),
            pltpu.VMEM((G,), jnp.int32),
            pltpu.VMEM((G, 128), jnp.float32),
            pltpu.VMEM((G, 128), jnp.float32),
            pltpu.SemaphoreType.DMA,
            pltpu.SemaphoreType.DMA,
        ],
    )
    def k(y_hbm, d0_hbm, d1_hbm, w0_hbm, w1_hbm, out_hbm,
          buf0, buf1, obuf, idx0_v, idx1_v, wv0, wv1, sem0, sem1):
        wid = lax.axis_index("s") * 2 + lax.axis_index("c")
        base = wid * per_w

        def group(gi, _):
            off = base + gi * G
            pltpu.sync_copy(d0_hbm.at[pl.ds(off, G)], idx0_v)
            pltpu.sync_copy(d1_hbm.at[pl.ds(off, G)], idx1_v)
            pltpu.sync_copy(w0_hbm.at[pl.ds(off, G)], wv0)
            pltpu.sync_copy(w1_hbm.at[pl.ds(off, G)], wv1)
            a = pltpu.async_copy(y_hbm.at[idx0_v], buf0, sem0)
            b = pltpu.async_copy(y_hbm.at[idx1_v], buf1, sem1)
            a.wait()
            b.wait()

            def token(t, _):
                a0 = wv0[t, pl.ds(0, 16)]
                a1 = wv1[t, pl.ds(0, 16)]

                def chunk(c, _):
                    y0 = buf0[t, pl.ds(c * 16, 16)]
                    y1 = buf1[t, pl.ds(c * 16, 16)]
                    obuf[t, pl.ds(c * 16, 16)] = a0 * y0 + a1 * y1
                    return 0

                lax.fori_loop(0, DIM // 16, chunk, 0)
                return 0

            lax.fori_loop(0, G, token, 0)
            pltpu.sync_copy(obuf, out_hbm.at[pl.ds(off, G)])
            return 0

        lax.fori_loop(0, ngroups, group, 0)

    return k(y, d0, d1, w0a, w1a)


def kernel(x, gate_w, Wg, Wu, Wd):
    bsz, seqlen, dim = x.shape
    T = bsz * seqlen
    S = 2 * T + NE * TILE
    flat = x.reshape(T, dim)

    xb, w0a, w1a, d0a, d1a, te_arr = _run_tc_stages(flat, gate_w)
    d0 = d0a[:, 0]
    d1 = d1a[:, 0]
    te = te_arr[0, :S // TILE]

    xbi = lax.bitcast_convert_type(xb.reshape(T, DIM // 2, 2), jnp.int32)
    xdi = _sc_dispatch(xbi, d0, d1, S)
    xd = lax.bitcast_convert_type(xdi, jnp.bfloat16).reshape(S, DIM)

    y = _run_mlp(xd, te, Wg.astype(jnp.bfloat16), Wu.astype(jnp.bfloat16),
                 Wd.astype(jnp.bfloat16))

    out = _sc_combine(y, d0, d1, w0a, w1a)
    return out.reshape(bsz, seqlen, dim)


# speedup vs baseline: 2.1205x; 2.1205x over previous
import functools

import jax
import jax.numpy as jnp
from jax import lax
from jax.experimental import pallas as pl
from jax.experimental.pallas import tpu as pltpu
from jax.experimental.pallas import tpu_sc as plsc

DIM = 2048
HID = 1024
NE = 8
TB = 512
TILE = 256


def _router_kernel(x_ref, gwt_ref, i0_ref, i1_ref, w0_ref, w1_ref,
                   r0_ref, r1_ref, counts_ref):
    i = pl.program_id(0)

    @pl.when(i == 0)
    def _():
        counts_ref[...] = jnp.zeros_like(counts_ref)

    x = x_ref[...]
    xb = x.astype(jnp.bfloat16)
    logits = jnp.dot(xb, gwt_ref[...].astype(jnp.bfloat16),
                     preferred_element_type=jnp.float32)
    lane = lax.broadcasted_iota(jnp.int32, (TB, 128), 1)
    neg = jnp.float32(-1e30)
    logits = jnp.where(lane < NE, logits, neg)
    m0 = jnp.max(logits, axis=1, keepdims=True)
    i0 = jnp.min(jnp.where(logits == m0, lane, 127), axis=1, keepdims=True)
    logits1 = jnp.where(lane == i0, neg, logits)
    m1 = jnp.max(logits1, axis=1, keepdims=True)
    i1 = jnp.min(jnp.where(logits1 == m1, lane, 127), axis=1, keepdims=True)
    s0 = jax.nn.sigmoid(m0)
    s1 = jax.nn.sigmoid(m1)
    denom = s0 + s1 + jnp.float32(1e-9)

    oh = ((lane == i0) | (lane == i1)).astype(jnp.float32)
    row = lax.broadcasted_iota(jnp.int32, (TB, TB), 0)
    col = lax.broadcasted_iota(jnp.int32, (TB, TB), 1)
    L = (row > col).astype(jnp.float32)
    ranks = jnp.dot(L, oh, preferred_element_type=jnp.float32) + counts_ref[...]
    r0 = jnp.sum(jnp.where(lane == i0, ranks, 0.0), axis=1, keepdims=True)
    r1 = jnp.sum(jnp.where(lane == i1, ranks, 0.0), axis=1, keepdims=True)
    counts_ref[...] += jnp.sum(oh, axis=0, keepdims=True)

    i0_ref[...] = jnp.broadcast_to(i0, (TB, 128))
    i1_ref[...] = jnp.broadcast_to(i1, (TB, 128))
    w0_ref[...] = jnp.broadcast_to(s0 / denom, (TB, 128))
    w1_ref[...] = jnp.broadcast_to(s1 / denom, (TB, 128))
    r0_ref[...] = jnp.broadcast_to(r0, (TB, 128)).astype(jnp.int32)
    r1_ref[...] = jnp.broadcast_to(r1, (TB, 128)).astype(jnp.int32)


def _dest_kernel(counts_ref, i0_ref, i1_ref, r0_ref, r1_ref,
                 d0_ref, d1_ref, te_ref):
    counts = counts_ref[...]
    pc = jnp.ceil(counts * (1.0 / TILE)) * TILE
    row = lax.broadcasted_iota(jnp.int32, (128, 128), 0)
    col = lax.broadcasted_iota(jnp.int32, (128, 128), 1)
    LT = (row < col).astype(jnp.float32)
    offs = jnp.dot(pc, LT, preferred_element_type=jnp.float32)
    lane = lax.broadcasted_iota(jnp.int32, (TB, 128), 1)
    offs_b = jnp.broadcast_to(offs, (TB, 128))
    o0 = jnp.sum(jnp.where(lane == i0_ref[...], offs_b, 0.0), axis=1,
                 keepdims=True)
    o1 = jnp.sum(jnp.where(lane == i1_ref[...], offs_b, 0.0), axis=1,
                 keepdims=True)
    d0 = o0.astype(jnp.int32) + r0_ref[:, :1]
    d1 = o1.astype(jnp.int32) + r1_ref[:, :1]
    d0_ref[...] = d0.reshape(1, 1, TB)
    d1_ref[...] = d1.reshape(1, 1, TB)

    @pl.when(pl.program_id(0) == 0)
    def _():
        bounds = offs + pc
        lane1 = lax.broadcasted_iota(jnp.int32, (1, 128), 1)
        te = jnp.zeros((1, 128), jnp.float32)
        for e in range(NE):
            be = jnp.sum(jnp.where(lane1 == e, bounds, 0.0), axis=1,
                         keepdims=True)
            te += (jnp.broadcast_to(be, (1, 128)) <=
                   (lane1 * TILE).astype(jnp.float32)).astype(jnp.float32)
        te_ref[...] = jnp.minimum(te, NE - 1).astype(jnp.int32)


def _mlp_kernel(te_ref, xd_ref, wg_ref, wu_ref, wd_ref, y_ref):
    xb = xd_ref[...].astype(jnp.bfloat16)
    dn = (((1,), (1,)), ((), ()))
    g = lax.dot_general(xb, wg_ref[0], dn, preferred_element_type=jnp.float32)
    u = lax.dot_general(xb, wu_ref[0], dn, preferred_element_type=jnp.float32)
    h = (g * jax.nn.sigmoid(g)) * u
    y_ref[...] = lax.dot_general(h.astype(jnp.bfloat16), wd_ref[0], dn,
                                 preferred_element_type=jnp.float32)


def _run_tc_stages(flat, gate_w):
    T = flat.shape[0]
    nb = T // TB
    gwt = jnp.zeros((DIM, 128), jnp.float32).at[:, :NE].set(gate_w.T)

    outs = pl.pallas_call(
        _router_kernel,
        grid=(nb,),
        in_specs=[
            pl.BlockSpec((TB, DIM), lambda i: (i, 0)),
            pl.BlockSpec((DIM, 128), lambda i: (0, 0)),
        ],
        out_specs=[
            pl.BlockSpec((TB, 128), lambda i: (i, 0)),
            pl.BlockSpec((TB, 128), lambda i: (i, 0)),
            pl.BlockSpec((TB, 128), lambda i: (i, 0)),
            pl.BlockSpec((TB, 128), lambda i: (i, 0)),
            pl.BlockSpec((TB, 128), lambda i: (i, 0)),
            pl.BlockSpec((TB, 128), lambda i: (i, 0)),
            pl.BlockSpec((1, 128), lambda i: (0, 0)),
        ],
        out_shape=[
            jax.ShapeDtypeStruct((T, 128), jnp.int32),
            jax.ShapeDtypeStruct((T, 128), jnp.int32),
            jax.ShapeDtypeStruct((T, 128), jnp.float32),
            jax.ShapeDtypeStruct((T, 128), jnp.float32),
            jax.ShapeDtypeStruct((T, 128), jnp.int32),
            jax.ShapeDtypeStruct((T, 128), jnp.int32),
            jax.ShapeDtypeStruct((1, 128), jnp.float32),
        ],
    )(flat, gwt)
    i0a, i1a, w0a, w1a, r0a, r1a, counts = outs

    d0a, d1a, te_arr = pl.pallas_call(
        _dest_kernel,
        grid=(nb,),
        in_specs=[
            pl.BlockSpec((1, 128), lambda i: (0, 0)),
            pl.BlockSpec((TB, 128), lambda i: (i, 0)),
            pl.BlockSpec((TB, 128), lambda i: (i, 0)),
            pl.BlockSpec((TB, 128), lambda i: (i, 0)),
            pl.BlockSpec((TB, 128), lambda i: (i, 0)),
        ],
        out_specs=[
            pl.BlockSpec((1, 1, TB), lambda i: (i, 0, 0)),
            pl.BlockSpec((1, 1, TB), lambda i: (i, 0, 0)),
            pl.BlockSpec((1, 128), lambda i: (0, 0)),
        ],
        out_shape=[
            jax.ShapeDtypeStruct((nb, 1, TB), jnp.int32),
            jax.ShapeDtypeStruct((nb, 1, TB), jnp.int32),
            jax.ShapeDtypeStruct((1, 128), jnp.int32),
        ],
    )(counts, i0a, i1a, r0a, r1a)
    return w0a, w1a, d0a, d1a, te_arr


def _run_mlp(xd, te, Wg_b, Wu_b, Wd_b):
    S = xd.shape[0]
    ntiles = S // TILE
    grid_spec = pltpu.PrefetchScalarGridSpec(
        num_scalar_prefetch=1,
        grid=(ntiles,),
        in_specs=[
            pl.BlockSpec((TILE, DIM), lambda i, te_r: (i, 0)),
            pl.BlockSpec((1, HID, DIM), lambda i, te_r: (te_r[i], 0, 0)),
            pl.BlockSpec((1, HID, DIM), lambda i, te_r: (te_r[i], 0, 0)),
            pl.BlockSpec((1, DIM, HID), lambda i, te_r: (te_r[i], 0, 0)),
        ],
        out_specs=pl.BlockSpec((TILE, DIM), lambda i, te_r: (i, 0)),
    )
    return pl.pallas_call(
        _mlp_kernel,
        grid_spec=grid_spec,
        out_shape=jax.ShapeDtypeStruct((S, DIM), jnp.float32),
    )(te, xd, Wg_b, Wu_b, Wd_b)


def _sc_dispatch(flat, d0, d1, S):
    T = flat.shape[0]
    NW = 32
    per_w = T // NW
    ROWS = 32
    ngroups = per_w // ROWS
    mesh = plsc.VectorSubcoreMesh(core_axis_name="c", subcore_axis_name="s")

    @functools.partial(
        pl.kernel, mesh=mesh,
        out_type=jax.ShapeDtypeStruct((S, DIM), jnp.float32),
        scratch_types=[
            pltpu.VMEM((ROWS, DIM), jnp.float32),
            pltpu.VMEM((ROWS,), jnp.int32),
            pltpu.VMEM((ROWS,), jnp.int32),
            pltpu.SemaphoreType.DMA,
            pltpu.SemaphoreType.DMA,
        ],
    )
    def k(x_hbm, d0_hbm, d1_hbm, xd_hbm, rows_v, idx0_v, idx1_v, sem0, sem1):
        wid = lax.axis_index("s") * 2 + lax.axis_index("c")
        base = wid * per_w
        for g in range(ngroups):
            off = base + g * ROWS
            pltpu.sync_copy(x_hbm.at[pl.ds(off, ROWS)], rows_v)
            pltpu.sync_copy(d0_hbm.at[pl.ds(off, ROWS)], idx0_v)
            pltpu.sync_copy(d1_hbm.at[pl.ds(off, ROWS)], idx1_v)
            a = pltpu.async_copy(rows_v, xd_hbm.at[idx0_v], sem0)
            b = pltpu.async_copy(rows_v, xd_hbm.at[idx1_v], sem1)
            a.wait()
            b.wait()

    return k(flat, d0, d1)


def _sc_combine(y, d0, d1, w0a, w1a):
    T = d0.shape[0]
    NW = 32
    per_w = T // NW
    G = 16
    ngroups = per_w // G
    mesh = plsc.VectorSubcoreMesh(core_axis_name="c", subcore_axis_name="s")

    @functools.partial(
        pl.kernel, mesh=mesh,
        out_type=jax.ShapeDtypeStruct((T, DIM), jnp.float32),
        scratch_types=[
            pltpu.VMEM((G, DIM), jnp.float32),
            pltpu.VMEM((G, DIM), jnp.float32),
            pltpu.VMEM((G, DIM), jnp.float32),
            pltpu.VMEM((G,), jnp.int32),
            pltpu.VMEM((G,), jnp.int32),
            pltpu.VMEM((G, 128), jnp.float32),
            pltpu.VMEM((G, 128), jnp.float32),
            pltpu.SemaphoreType.DMA,
            pltpu.SemaphoreType.DMA,
        ],
    )
    def k(y_hbm, d0_hbm, d1_hbm, w0_hbm, w1_hbm, out_hbm,
          buf0, buf1, obuf, idx0_v, idx1_v, wv0, wv1, sem0, sem1):
        wid = lax.axis_index("s") * 2 + lax.axis_index("c")
        base = wid * per_w

        def group(gi, _):
            off = base + gi * G
            pltpu.sync_copy(d0_hbm.at[pl.ds(off, G)], idx0_v)
            pltpu.sync_copy(d1_hbm.at[pl.ds(off, G)], idx1_v)
            pltpu.sync_copy(w0_hbm.at[pl.ds(off, G)], wv0)
            pltpu.sync_copy(w1_hbm.at[pl.ds(off, G)], wv1)
            a = pltpu.async_copy(y_hbm.at[idx0_v], buf0, sem0)
            b = pltpu.async_copy(y_hbm.at[idx1_v], buf1, sem1)
            a.wait()
            b.wait()

            def token(t, _):
                a0 = wv0[t, pl.ds(0, 16)]
                a1 = wv1[t, pl.ds(0, 16)]

                def chunk(c, _):
                    y0 = buf0[t, pl.ds(c * 16, 16)]
                    y1 = buf1[t, pl.ds(c * 16, 16)]
                    obuf[t, pl.ds(c * 16, 16)] = a0 * y0 + a1 * y1
                    return 0

                lax.fori_loop(0, DIM // 16, chunk, 0, unroll=8)
                return 0

            lax.fori_loop(0, G, token, 0)
            pltpu.sync_copy(obuf, out_hbm.at[pl.ds(off, G)])
            return 0

        lax.fori_loop(0, ngroups, group, 0)

    return k(y, d0, d1, w0a, w1a)


def kernel(x, gate_w, Wg, Wu, Wd):
    bsz, seqlen, dim = x.shape
    T = bsz * seqlen
    S = 2 * T + NE * TILE
    flat = x.reshape(T, dim)

    w0a, w1a, d0a, d1a, te_arr = _run_tc_stages(flat, gate_w)
    d0 = d0a.reshape(T)
    d1 = d1a.reshape(T)
    te = te_arr[0, :S // TILE]

    xd = _sc_dispatch(flat, d0, d1, S)

    y = _run_mlp(xd, te, Wg.astype(jnp.bfloat16), Wu.astype(jnp.bfloat16),
                 Wd.astype(jnp.bfloat16))

    out = _sc_combine(y, d0, d1, w0a, w1a)
    return out.reshape(bsz, seqlen, dim)


# speedup vs baseline: 2.4299x; 1.1459x over previous
import functools

import jax
import jax.numpy as jnp
from jax import lax
from jax.experimental import pallas as pl
from jax.experimental.pallas import tpu as pltpu
from jax.experimental.pallas import tpu_sc as plsc

DIM = 2048
HID = 1024
NE = 8
TB = 512
TILE = 256


def _router_kernel(x_ref, gwt_ref, xp_ref, i0_ref, i1_ref, w0_ref, w1_ref,
                   r0_ref, r1_ref, counts_ref):
    i = pl.program_id(0)

    @pl.when(i == 0)
    def _():
        counts_ref[...] = jnp.zeros_like(counts_ref)

    x = x_ref[...]
    xb = x.astype(jnp.bfloat16)
    lo = lax.bitcast_convert_type(xb[:, :DIM // 2], jnp.uint16)
    hi = lax.bitcast_convert_type(xb[:, DIM // 2:], jnp.uint16)
    packed = lo.astype(jnp.uint32) | (hi.astype(jnp.uint32) << 16)
    xp_ref[...] = lax.bitcast_convert_type(packed, jnp.int32)
    logits = jnp.dot(xb, gwt_ref[...].astype(jnp.bfloat16),
                     preferred_element_type=jnp.float32)
    lane = lax.broadcasted_iota(jnp.int32, (TB, 128), 1)
    neg = jnp.float32(-1e30)
    logits = jnp.where(lane < NE, logits, neg)
    m0 = jnp.max(logits, axis=1, keepdims=True)
    i0 = jnp.min(jnp.where(logits == m0, lane, 127), axis=1, keepdims=True)
    logits1 = jnp.where(lane == i0, neg, logits)
    m1 = jnp.max(logits1, axis=1, keepdims=True)
    i1 = jnp.min(jnp.where(logits1 == m1, lane, 127), axis=1, keepdims=True)
    s0 = jax.nn.sigmoid(m0)
    s1 = jax.nn.sigmoid(m1)
    denom = s0 + s1 + jnp.float32(1e-9)

    oh = ((lane == i0) | (lane == i1)).astype(jnp.float32)
    row = lax.broadcasted_iota(jnp.int32, (TB, TB), 0)
    col = lax.broadcasted_iota(jnp.int32, (TB, TB), 1)
    L = (row > col).astype(jnp.float32)
    ranks = jnp.dot(L, oh, preferred_element_type=jnp.float32) + counts_ref[...]
    r0 = jnp.sum(jnp.where(lane == i0, ranks, 0.0), axis=1, keepdims=True)
    r1 = jnp.sum(jnp.where(lane == i1, ranks, 0.0), axis=1, keepdims=True)
    counts_ref[...] += jnp.sum(oh, axis=0, keepdims=True)

    i0_ref[...] = jnp.broadcast_to(i0, (TB, 128))
    i1_ref[...] = jnp.broadcast_to(i1, (TB, 128))
    w0_ref[...] = jnp.broadcast_to(s0 / denom, (TB, 128))
    w1_ref[...] = jnp.broadcast_to(s1 / denom, (TB, 128))
    r0_ref[...] = jnp.broadcast_to(r0, (TB, 128)).astype(jnp.int32)
    r1_ref[...] = jnp.broadcast_to(r1, (TB, 128)).astype(jnp.int32)


def _dest_kernel(counts_ref, i0_ref, i1_ref, r0_ref, r1_ref,
                 d0_ref, d1_ref, te_ref):
    counts = counts_ref[...]
    pc = jnp.ceil(counts * (1.0 / TILE)) * TILE
    row = lax.broadcasted_iota(jnp.int32, (128, 128), 0)
    col = lax.broadcasted_iota(jnp.int32, (128, 128), 1)
    LT = (row < col).astype(jnp.float32)
    offs = jnp.dot(pc, LT, preferred_element_type=jnp.float32)
    lane = lax.broadcasted_iota(jnp.int32, (TB, 128), 1)
    offs_b = jnp.broadcast_to(offs, (TB, 128))
    o0 = jnp.sum(jnp.where(lane == i0_ref[...], offs_b, 0.0), axis=1,
                 keepdims=True)
    o1 = jnp.sum(jnp.where(lane == i1_ref[...], offs_b, 0.0), axis=1,
                 keepdims=True)
    d0 = o0.astype(jnp.int32) + r0_ref[:, :1]
    d1 = o1.astype(jnp.int32) + r1_ref[:, :1]
    d0_ref[...] = d0.reshape(1, 1, TB)
    d1_ref[...] = d1.reshape(1, 1, TB)

    @pl.when(pl.program_id(0) == 0)
    def _():
        bounds = offs + pc
        lane1 = lax.broadcasted_iota(jnp.int32, (1, 128), 1)
        te = jnp.zeros((1, 128), jnp.float32)
        for e in range(NE):
            be = jnp.sum(jnp.where(lane1 == e, bounds, 0.0), axis=1,
                         keepdims=True)
            te += (jnp.broadcast_to(be, (1, 128)) <=
                   (lane1 * TILE).astype(jnp.float32)).astype(jnp.float32)
        te_ref[...] = jnp.minimum(te, NE - 1).astype(jnp.int32)


def _mlp_kernel(te_ref, xd_ref, wg_ref, wu_ref, wd_ref, y_ref):
    packed = lax.bitcast_convert_type(xd_ref[...], jnp.uint32)
    lo = lax.bitcast_convert_type((packed & 0xFFFF).astype(jnp.uint16),
                                  jnp.bfloat16)
    hi = lax.bitcast_convert_type((packed >> 16).astype(jnp.uint16),
                                  jnp.bfloat16)
    xb = jnp.concatenate([lo, hi], axis=1)
    dn = (((1,), (1,)), ((), ()))
    g = lax.dot_general(xb, wg_ref[0], dn, preferred_element_type=jnp.float32)
    u = lax.dot_general(xb, wu_ref[0], dn, preferred_element_type=jnp.float32)
    h = (g * jax.nn.sigmoid(g)) * u
    y_ref[...] = lax.dot_general(h.astype(jnp.bfloat16), wd_ref[0], dn,
                                 preferred_element_type=jnp.float32)


def _run_tc_stages(flat, gate_w):
    T = flat.shape[0]
    nb = T // TB
    gwt = jnp.zeros((DIM, 128), jnp.float32).at[:, :NE].set(gate_w.T)

    outs = pl.pallas_call(
        _router_kernel,
        grid=(nb,),
        in_specs=[
            pl.BlockSpec((TB, DIM), lambda i: (i, 0)),
            pl.BlockSpec((DIM, 128), lambda i: (0, 0)),
        ],
        out_specs=[
            pl.BlockSpec((TB, DIM // 2), lambda i: (i, 0)),
            pl.BlockSpec((TB, 128), lambda i: (i, 0)),
            pl.BlockSpec((TB, 128), lambda i: (i, 0)),
            pl.BlockSpec((TB, 128), lambda i: (i, 0)),
            pl.BlockSpec((TB, 128), lambda i: (i, 0)),
            pl.BlockSpec((TB, 128), lambda i: (i, 0)),
            pl.BlockSpec((TB, 128), lambda i: (i, 0)),
            pl.BlockSpec((1, 128), lambda i: (0, 0)),
        ],
        out_shape=[
            jax.ShapeDtypeStruct((T, DIM // 2), jnp.int32),
            jax.ShapeDtypeStruct((T, 128), jnp.int32),
            jax.ShapeDtypeStruct((T, 128), jnp.int32),
            jax.ShapeDtypeStruct((T, 128), jnp.float32),
            jax.ShapeDtypeStruct((T, 128), jnp.float32),
            jax.ShapeDtypeStruct((T, 128), jnp.int32),
            jax.ShapeDtypeStruct((T, 128), jnp.int32),
            jax.ShapeDtypeStruct((1, 128), jnp.float32),
        ],
    )(flat, gwt)
    xpk, i0a, i1a, w0a, w1a, r0a, r1a, counts = outs

    d0a, d1a, te_arr = pl.pallas_call(
        _dest_kernel,
        grid=(nb,),
        in_specs=[
            pl.BlockSpec((1, 128), lambda i: (0, 0)),
            pl.BlockSpec((TB, 128), lambda i: (i, 0)),
            pl.BlockSpec((TB, 128), lambda i: (i, 0)),
            pl.BlockSpec((TB, 128), lambda i: (i, 0)),
            pl.BlockSpec((TB, 128), lambda i: (i, 0)),
        ],
        out_specs=[
            pl.BlockSpec((1, 1, TB), lambda i: (i, 0, 0)),
            pl.BlockSpec((1, 1, TB), lambda i: (i, 0, 0)),
            pl.BlockSpec((1, 128), lambda i: (0, 0)),
        ],
        out_shape=[
            jax.ShapeDtypeStruct((nb, 1, TB), jnp.int32),
            jax.ShapeDtypeStruct((nb, 1, TB), jnp.int32),
            jax.ShapeDtypeStruct((1, 128), jnp.int32),
        ],
    )(counts, i0a, i1a, r0a, r1a)
    return xpk, w0a, w1a, d0a, d1a, te_arr


def _run_mlp(xd, te, Wg_b, Wu_b, Wd_b):
    S = xd.shape[0]
    ntiles = S // TILE
    grid_spec = pltpu.PrefetchScalarGridSpec(
        num_scalar_prefetch=1,
        grid=(ntiles,),
        in_specs=[
            pl.BlockSpec((TILE, DIM // 2), lambda i, te_r: (i, 0)),
            pl.BlockSpec((1, HID, DIM), lambda i, te_r: (te_r[i], 0, 0)),
            pl.BlockSpec((1, HID, DIM), lambda i, te_r: (te_r[i], 0, 0)),
            pl.BlockSpec((1, DIM, HID), lambda i, te_r: (te_r[i], 0, 0)),
        ],
        out_specs=pl.BlockSpec((TILE, DIM), lambda i, te_r: (i, 0)),
    )
    return pl.pallas_call(
        _mlp_kernel,
        grid_spec=grid_spec,
        out_shape=jax.ShapeDtypeStruct((S, DIM), jnp.float32),
    )(te, xd, Wg_b, Wu_b, Wd_b)


def _sc_dispatch(xpk, d0, d1, S):
    T = xpk.shape[0]
    NW = 32
    per_w = T // NW
    ROWS = 64
    ngroups = per_w // ROWS
    mesh = plsc.VectorSubcoreMesh(core_axis_name="c", subcore_axis_name="s")

    @functools.partial(
        pl.kernel, mesh=mesh,
        out_type=jax.ShapeDtypeStruct((S, DIM // 2), jnp.int32),
        scratch_types=[
            pltpu.VMEM((ROWS, DIM // 2), jnp.int32),
            pltpu.VMEM((ROWS,), jnp.int32),
            pltpu.VMEM((ROWS,), jnp.int32),
            pltpu.SemaphoreType.DMA,
            pltpu.SemaphoreType.DMA,
        ],
    )
    def k(x_hbm, d0_hbm, d1_hbm, xd_hbm, rows_v, idx0_v, idx1_v, sem0, sem1):
        wid = lax.axis_index("s") * 2 + lax.axis_index("c")
        base = wid * per_w
        for g in range(ngroups):
            off = base + g * ROWS
            pltpu.sync_copy(x_hbm.at[pl.ds(off, ROWS)], rows_v)
            pltpu.sync_copy(d0_hbm.at[pl.ds(off, ROWS)], idx0_v)
            pltpu.sync_copy(d1_hbm.at[pl.ds(off, ROWS)], idx1_v)
            a = pltpu.async_copy(rows_v, xd_hbm.at[idx0_v], sem0)
            b = pltpu.async_copy(rows_v, xd_hbm.at[idx1_v], sem1)
            a.wait()
            b.wait()

    return k(xpk, d0, d1)


def _sc_combine(y, d0, d1, w0a, w1a):
    T = d0.shape[0]
    NW = 32
    per_w = T // NW
    G = 8
    ngroups = per_w // G
    mesh = plsc.VectorSubcoreMesh(core_axis_name="c", subcore_axis_name="s")

    @functools.partial(
        pl.kernel, mesh=mesh,
        out_type=jax.ShapeDtypeStruct((T, DIM), jnp.float32),
        scratch_types=[
            [pltpu.VMEM((G, DIM), jnp.float32)] * 2,
            [pltpu.VMEM((G, DIM), jnp.float32)] * 2,
            [pltpu.VMEM((G, DIM), jnp.float32)] * 2,
            [pltpu.VMEM((G,), jnp.int32)] * 2,
            [pltpu.VMEM((G,), jnp.int32)] * 2,
            [pltpu.VMEM((G, 128), jnp.float32)] * 2,
            [pltpu.VMEM((G, 128), jnp.float32)] * 2,
            [pltpu.SemaphoreType.DMA] * 2,
            [pltpu.SemaphoreType.DMA] * 2,
            [pltpu.SemaphoreType.DMA] * 2,
        ],
    )
    def k(y_hbm, d0_hbm, d1_hbm, w0_hbm, w1_hbm, out_hbm,
          buf0, buf1, obuf, idx0_v, idx1_v, wv0, wv1, gsem0, gsem1, osem):
        wid = lax.axis_index("s") * 2 + lax.axis_index("c")
        base = wid * per_w

        def start_gather(gi, p):
            off = base + gi * G
            pltpu.sync_copy(d0_hbm.at[pl.ds(off, G)], idx0_v[p])
            pltpu.sync_copy(d1_hbm.at[pl.ds(off, G)], idx1_v[p])
            pltpu.sync_copy(w0_hbm.at[pl.ds(off, G)], wv0[p])
            pltpu.sync_copy(w1_hbm.at[pl.ds(off, G)], wv1[p])
            pltpu.async_copy(y_hbm.at[idx0_v[p]], buf0[p], gsem0[p])
            pltpu.async_copy(y_hbm.at[idx1_v[p]], buf1[p], gsem1[p])

        def finish_group(gi, p):
            off = base + gi * G
            pltpu.make_async_copy(y_hbm.at[idx0_v[p]], buf0[p],
                                  gsem0[p]).wait()
            pltpu.make_async_copy(y_hbm.at[idx1_v[p]], buf1[p],
                                  gsem1[p]).wait()

            def token(t, _):
                a0 = wv0[p][t, pl.ds(0, 16)]
                a1 = wv1[p][t, pl.ds(0, 16)]

                def chunk(c, _):
                    y0 = buf0[p][t, pl.ds(c * 16, 16)]
                    y1 = buf1[p][t, pl.ds(c * 16, 16)]
                    obuf[p][t, pl.ds(c * 16, 16)] = a0 * y0 + a1 * y1
                    return 0

                lax.fori_loop(0, DIM // 16, chunk, 0, unroll=2)
                return 0

            lax.fori_loop(0, G, token, 0)
            pltpu.async_copy(obuf[p], out_hbm.at[pl.ds(off, G)], osem[p])

        def wait_out(gi, p):
            off = base + gi * G
            pltpu.make_async_copy(obuf[p], out_hbm.at[pl.ds(off, G)],
                                  osem[p]).wait()

        start_gather(0, 0)

        def pair(i, _):
            g = i * 2
            start_gather(g + 1, 1)

            @pl.when(i > 0)
            def _():
                wait_out(g - 2, 0)
            finish_group(g, 0)

            @pl.when(i < ngroups // 2 - 1)
            def _():
                start_gather(g + 2, 0)

            @pl.when(i > 0)
            def _():
                wait_out(g - 1, 1)
            finish_group(g + 1, 1)
            return 0

        lax.fori_loop(0, ngroups // 2, pair, 0)
        wait_out(ngroups - 2, 0)
        wait_out(ngroups - 1, 1)

    return k(y, d0, d1, w0a, w1a)


def _half_pipeline(flat_h, gate_w, Wg_b, Wu_b, Wd_b):
    Th = flat_h.shape[0]
    S = 2 * Th + NE * TILE
    xpk, w0a, w1a, d0a, d1a, te_arr = _run_tc_stages(flat_h, gate_w)
    d0 = d0a.reshape(Th)
    d1 = d1a.reshape(Th)
    te = te_arr[0, :S // TILE]
    xd = _sc_dispatch(xpk, d0, d1, S)
    y = _run_mlp(xd, te, Wg_b, Wu_b, Wd_b)
    return _sc_combine(y, d0, d1, w0a, w1a)


def kernel(x, gate_w, Wg, Wu, Wd):
    bsz, seqlen, dim = x.shape
    T = bsz * seqlen
    flat = x.reshape(T, dim)
    Wg_b = Wg.astype(jnp.bfloat16)
    Wu_b = Wu.astype(jnp.bfloat16)
    Wd_b = Wd.astype(jnp.bfloat16)
    out_a = _half_pipeline(flat[:T // 2], gate_w, Wg_b, Wu_b, Wd_b)
    out_b = _half_pipeline(flat[T // 2:], gate_w, Wg_b, Wu_b, Wd_b)
    return jnp.concatenate([out_a, out_b], axis=0).reshape(bsz, seqlen, dim)


# speedup vs baseline: 2.6866x; 1.1056x over previous
import functools

import jax
import jax.numpy as jnp
from jax import lax
from jax.experimental import pallas as pl
from jax.experimental.pallas import tpu as pltpu
from jax.experimental.pallas import tpu_sc as plsc

DIM = 2048
HID = 1024
NE = 8
TB = 512
TILE = 256


def _router_kernel(x_ref, gwt_ref, xp_ref, i0_ref, i1_ref, w0_ref, w1_ref,
                   r0_ref, r1_ref, counts_ref):
    i = pl.program_id(0)

    @pl.when(i == 0)
    def _():
        counts_ref[...] = jnp.zeros_like(counts_ref)

    x = x_ref[...]
    xb = x.astype(jnp.bfloat16)
    lo = lax.bitcast_convert_type(xb[:, :DIM // 2], jnp.uint16)
    hi = lax.bitcast_convert_type(xb[:, DIM // 2:], jnp.uint16)
    packed = lo.astype(jnp.uint32) | (hi.astype(jnp.uint32) << 16)
    xp_ref[...] = lax.bitcast_convert_type(packed, jnp.int32)
    logits = jnp.dot(xb, gwt_ref[...].astype(jnp.bfloat16),
                     preferred_element_type=jnp.float32)
    lane = lax.broadcasted_iota(jnp.int32, (TB, 128), 1)
    neg = jnp.float32(-1e30)
    logits = jnp.where(lane < NE, logits, neg)
    m0 = jnp.max(logits, axis=1, keepdims=True)
    i0 = jnp.min(jnp.where(logits == m0, lane, 127), axis=1, keepdims=True)
    logits1 = jnp.where(lane == i0, neg, logits)
    m1 = jnp.max(logits1, axis=1, keepdims=True)
    i1 = jnp.min(jnp.where(logits1 == m1, lane, 127), axis=1, keepdims=True)
    s0 = jax.nn.sigmoid(m0)
    s1 = jax.nn.sigmoid(m1)
    denom = s0 + s1 + jnp.float32(1e-9)

    oh = ((lane == i0) | (lane == i1)).astype(jnp.float32)
    row = lax.broadcasted_iota(jnp.int32, (TB, TB), 0)
    col = lax.broadcasted_iota(jnp.int32, (TB, TB), 1)
    L = (row > col).astype(jnp.float32)
    ranks = jnp.dot(L, oh, preferred_element_type=jnp.float32) + counts_ref[...]
    r0 = jnp.sum(jnp.where(lane == i0, ranks, 0.0), axis=1, keepdims=True)
    r1 = jnp.sum(jnp.where(lane == i1, ranks, 0.0), axis=1, keepdims=True)
    counts_ref[...] += jnp.sum(oh, axis=0, keepdims=True)

    i0_ref[...] = jnp.broadcast_to(i0, (TB, 128))
    i1_ref[...] = jnp.broadcast_to(i1, (TB, 128))
    w0_ref[...] = jnp.broadcast_to(s0 / denom, (TB, 128))
    w1_ref[...] = jnp.broadcast_to(s1 / denom, (TB, 128))
    r0_ref[...] = jnp.broadcast_to(r0, (TB, 128)).astype(jnp.int32)
    r1_ref[...] = jnp.broadcast_to(r1, (TB, 128)).astype(jnp.int32)


def _dest_kernel(counts_ref, i0_ref, i1_ref, r0_ref, r1_ref,
                 d0_ref, d1_ref, te_ref):
    counts = counts_ref[...]
    pc = jnp.ceil(counts * (1.0 / TILE)) * TILE
    row = lax.broadcasted_iota(jnp.int32, (128, 128), 0)
    col = lax.broadcasted_iota(jnp.int32, (128, 128), 1)
    LT = (row < col).astype(jnp.float32)
    offs = jnp.dot(pc, LT, preferred_element_type=jnp.float32)
    lane = lax.broadcasted_iota(jnp.int32, (TB, 128), 1)
    offs_b = jnp.broadcast_to(offs, (TB, 128))
    o0 = jnp.sum(jnp.where(lane == i0_ref[...], offs_b, 0.0), axis=1,
                 keepdims=True)
    o1 = jnp.sum(jnp.where(lane == i1_ref[...], offs_b, 0.0), axis=1,
                 keepdims=True)
    d0 = o0.astype(jnp.int32) + r0_ref[:, :1]
    d1 = o1.astype(jnp.int32) + r1_ref[:, :1]
    d0_ref[...] = d0.reshape(1, 1, TB)
    d1_ref[...] = d1.reshape(1, 1, TB)

    @pl.when(pl.program_id(0) == 0)
    def _():
        bounds = offs + pc
        lane1 = lax.broadcasted_iota(jnp.int32, (1, 128), 1)
        te = jnp.zeros((1, 128), jnp.float32)
        for e in range(NE):
            be = jnp.sum(jnp.where(lane1 == e, bounds, 0.0), axis=1,
                         keepdims=True)
            te += (jnp.broadcast_to(be, (1, 128)) <=
                   (lane1 * TILE).astype(jnp.float32)).astype(jnp.float32)
        te_ref[...] = jnp.minimum(te, NE - 1).astype(jnp.int32)


def _mlp_kernel(te_ref, xd_ref, wg_ref, wu_ref, wd_ref, y_ref):
    packed = lax.bitcast_convert_type(xd_ref[...], jnp.uint32)
    lo = lax.bitcast_convert_type((packed & 0xFFFF).astype(jnp.uint16),
                                  jnp.bfloat16)
    hi = lax.bitcast_convert_type((packed >> 16).astype(jnp.uint16),
                                  jnp.bfloat16)
    xb = jnp.concatenate([lo, hi], axis=1)
    dn = (((1,), (1,)), ((), ()))
    g = lax.dot_general(xb, wg_ref[0], dn, preferred_element_type=jnp.float32)
    u = lax.dot_general(xb, wu_ref[0], dn, preferred_element_type=jnp.float32)
    h = (g * jax.nn.sigmoid(g)) * u
    y_ref[...] = lax.dot_general(h.astype(jnp.bfloat16), wd_ref[0], dn,
                                 preferred_element_type=jnp.float32)


def _run_tc_stages(flat, gate_w):
    T = flat.shape[0]
    nb = T // TB
    gwt = jnp.zeros((DIM, 128), jnp.float32).at[:, :NE].set(gate_w.T)

    outs = pl.pallas_call(
        _router_kernel,
        grid=(nb,),
        in_specs=[
            pl.BlockSpec((TB, DIM), lambda i: (i, 0)),
            pl.BlockSpec((DIM, 128), lambda i: (0, 0)),
        ],
        out_specs=[
            pl.BlockSpec((TB, DIM // 2), lambda i: (i, 0)),
            pl.BlockSpec((TB, 128), lambda i: (i, 0)),
            pl.BlockSpec((TB, 128), lambda i: (i, 0)),
            pl.BlockSpec((TB, 128), lambda i: (i, 0)),
            pl.BlockSpec((TB, 128), lambda i: (i, 0)),
            pl.BlockSpec((TB, 128), lambda i: (i, 0)),
            pl.BlockSpec((TB, 128), lambda i: (i, 0)),
            pl.BlockSpec((1, 128), lambda i: (0, 0)),
        ],
        out_shape=[
            jax.ShapeDtypeStruct((T, DIM // 2), jnp.int32),
            jax.ShapeDtypeStruct((T, 128), jnp.int32),
            jax.ShapeDtypeStruct((T, 128), jnp.int32),
            jax.ShapeDtypeStruct((T, 128), jnp.float32),
            jax.ShapeDtypeStruct((T, 128), jnp.float32),
            jax.ShapeDtypeStruct((T, 128), jnp.int32),
            jax.ShapeDtypeStruct((T, 128), jnp.int32),
            jax.ShapeDtypeStruct((1, 128), jnp.float32),
        ],
    )(flat, gwt)
    xpk, i0a, i1a, w0a, w1a, r0a, r1a, counts = outs

    d0a, d1a, te_arr = pl.pallas_call(
        _dest_kernel,
        grid=(nb,),
        in_specs=[
            pl.BlockSpec((1, 128), lambda i: (0, 0)),
            pl.BlockSpec((TB, 128), lambda i: (i, 0)),
            pl.BlockSpec((TB, 128), lambda i: (i, 0)),
            pl.BlockSpec((TB, 128), lambda i: (i, 0)),
            pl.BlockSpec((TB, 128), lambda i: (i, 0)),
        ],
        out_specs=[
            pl.BlockSpec((1, 1, TB), lambda i: (i, 0, 0)),
            pl.BlockSpec((1, 1, TB), lambda i: (i, 0, 0)),
            pl.BlockSpec((1, 128), lambda i: (0, 0)),
        ],
        out_shape=[
            jax.ShapeDtypeStruct((nb, 1, TB), jnp.int32),
            jax.ShapeDtypeStruct((nb, 1, TB), jnp.int32),
            jax.ShapeDtypeStruct((1, 128), jnp.int32),
        ],
    )(counts, i0a, i1a, r0a, r1a)
    return xpk, w0a, w1a, d0a, d1a, te_arr


def _run_mlp(xd, te, Wg_b, Wu_b, Wd_b):
    S = xd.shape[0]
    ntiles = S // TILE
    grid_spec = pltpu.PrefetchScalarGridSpec(
        num_scalar_prefetch=1,
        grid=(ntiles,),
        in_specs=[
            pl.BlockSpec((TILE, DIM // 2), lambda i, te_r: (i, 0)),
            pl.BlockSpec((1, HID, DIM), lambda i, te_r: (te_r[i], 0, 0)),
            pl.BlockSpec((1, HID, DIM), lambda i, te_r: (te_r[i], 0, 0)),
            pl.BlockSpec((1, DIM, HID), lambda i, te_r: (te_r[i], 0, 0)),
        ],
        out_specs=pl.BlockSpec((TILE, DIM), lambda i, te_r: (i, 0)),
    )
    return pl.pallas_call(
        _mlp_kernel,
        grid_spec=grid_spec,
        out_shape=jax.ShapeDtypeStruct((S, DIM), jnp.float32),
    )(te, xd, Wg_b, Wu_b, Wd_b)


def _sc_dispatch(xpk, d0, d1, S):
    T = xpk.shape[0]
    NW = 32
    per_w = T // NW
    ROWS = 32
    ngroups = per_w // ROWS
    mesh = plsc.VectorSubcoreMesh(core_axis_name="c", subcore_axis_name="s")

    @functools.partial(
        pl.kernel, mesh=mesh,
        out_type=jax.ShapeDtypeStruct((S, DIM // 2), jnp.int32),
        scratch_types=[
            [pltpu.VMEM((ROWS, DIM // 2), jnp.int32)] * 2,
            [pltpu.VMEM((ROWS,), jnp.int32)] * 2,
            [pltpu.VMEM((ROWS,), jnp.int32)] * 2,
            [pltpu.SemaphoreType.DMA] * 2,
            pltpu.SemaphoreType.DMA,
            pltpu.SemaphoreType.DMA,
        ],
    )
    def k(x_hbm, d0_hbm, d1_hbm, xd_hbm, rows_v, idx0_v, idx1_v,
          insem, sem0, sem1):
        wid = lax.axis_index("s") * 2 + lax.axis_index("c")
        base = wid * per_w

        def start_in(g, p):
            off = base + g * ROWS
            pltpu.sync_copy(d0_hbm.at[pl.ds(off, ROWS)], idx0_v[p])
            pltpu.sync_copy(d1_hbm.at[pl.ds(off, ROWS)], idx1_v[p])
            pltpu.async_copy(x_hbm.at[pl.ds(off, ROWS)], rows_v[p], insem[p])

        start_in(0, 0)
        for g in range(ngroups):
            p = g % 2
            off = base + g * ROWS
            pltpu.make_async_copy(x_hbm.at[pl.ds(off, ROWS)], rows_v[p],
                                  insem[p]).wait()
            a = pltpu.async_copy(rows_v[p], xd_hbm.at[idx0_v[p]], sem0)
            b = pltpu.async_copy(rows_v[p], xd_hbm.at[idx1_v[p]], sem1)
            if g + 1 < ngroups:
                start_in(g + 1, 1 - p)
            a.wait()
            b.wait()

    return k(xpk, d0, d1)


def _sc_combine(y, d0, d1, w0a, w1a):
    T = d0.shape[0]
    NW = 32
    per_w = T // NW
    G = 8
    ngroups = per_w // G
    mesh = plsc.VectorSubcoreMesh(core_axis_name="c", subcore_axis_name="s")

    @functools.partial(
        pl.kernel, mesh=mesh,
        out_type=jax.ShapeDtypeStruct((T, DIM), jnp.float32),
        scratch_types=[
            [pltpu.VMEM((G, DIM), jnp.float32)] * 2,
            [pltpu.VMEM((G, DIM), jnp.float32)] * 2,
            [pltpu.VMEM((G, DIM), jnp.float32)] * 2,
            [pltpu.VMEM((G,), jnp.int32)] * 2,
            [pltpu.VMEM((G,), jnp.int32)] * 2,
            [pltpu.VMEM((G, 128), jnp.float32)] * 2,
            [pltpu.VMEM((G, 128), jnp.float32)] * 2,
            [pltpu.SemaphoreType.DMA] * 2,
            [pltpu.SemaphoreType.DMA] * 2,
            [pltpu.SemaphoreType.DMA] * 2,
        ],
    )
    def k(y_hbm, d0_hbm, d1_hbm, w0_hbm, w1_hbm, out_hbm,
          buf0, buf1, obuf, idx0_v, idx1_v, wv0, wv1, gsem0, gsem1, osem):
        wid = lax.axis_index("s") * 2 + lax.axis_index("c")
        base = wid * per_w

        def start_gather(gi, p):
            off = base + gi * G
            pltpu.sync_copy(d0_hbm.at[pl.ds(off, G)], idx0_v[p])
            pltpu.sync_copy(d1_hbm.at[pl.ds(off, G)], idx1_v[p])
            pltpu.sync_copy(w0_hbm.at[pl.ds(off, G)], wv0[p])
            pltpu.sync_copy(w1_hbm.at[pl.ds(off, G)], wv1[p])
            pltpu.async_copy(y_hbm.at[idx0_v[p]], buf0[p], gsem0[p])
            pltpu.async_copy(y_hbm.at[idx1_v[p]], buf1[p], gsem1[p])

        def finish_group(gi, p):
            off = base + gi * G
            pltpu.make_async_copy(y_hbm.at[idx0_v[p]], buf0[p],
                                  gsem0[p]).wait()
            pltpu.make_async_copy(y_hbm.at[idx1_v[p]], buf1[p],
                                  gsem1[p]).wait()

            def token(t, _):
                a0 = wv0[p][t, pl.ds(0, 16)]
                a1 = wv1[p][t, pl.ds(0, 16)]

                def chunk(c, _):
                    y0 = buf0[p][t, pl.ds(c * 16, 16)]
                    y1 = buf1[p][t, pl.ds(c * 16, 16)]
                    obuf[p][t, pl.ds(c * 16, 16)] = a0 * y0 + a1 * y1
                    return 0

                lax.fori_loop(0, DIM // 16, chunk, 0, unroll=2)
                return 0

            lax.fori_loop(0, G, token, 0)
            pltpu.async_copy(obuf[p], out_hbm.at[pl.ds(off, G)], osem[p])

        def wait_out(gi, p):
            off = base + gi * G
            pltpu.make_async_copy(obuf[p], out_hbm.at[pl.ds(off, G)],
                                  osem[p]).wait()

        start_gather(0, 0)

        def pair(i, _):
            g = i * 2
            start_gather(g + 1, 1)

            @pl.when(i > 0)
            def _():
                wait_out(g - 2, 0)
            finish_group(g, 0)

            @pl.when(i < ngroups // 2 - 1)
            def _():
                start_gather(g + 2, 0)

            @pl.when(i > 0)
            def _():
                wait_out(g - 1, 1)
            finish_group(g + 1, 1)
            return 0

        lax.fori_loop(0, ngroups // 2, pair, 0)
        wait_out(ngroups - 2, 0)
        wait_out(ngroups - 1, 1)

    return k(y, d0, d1, w0a, w1a)


def kernel(x, gate_w, Wg, Wu, Wd):
    bsz, seqlen, dim = x.shape
    T = bsz * seqlen
    S = 2 * T + NE * TILE
    flat = x.reshape(T, dim)

    xpk, w0a, w1a, d0a, d1a, te_arr = _run_tc_stages(flat, gate_w)
    d0 = d0a.reshape(T)
    d1 = d1a.reshape(T)
    te = te_arr[0, :S // TILE]

    xd = _sc_dispatch(xpk, d0, d1, S)

    y = _run_mlp(xd, te, Wg.astype(jnp.bfloat16), Wu.astype(jnp.bfloat16),
                 Wd.astype(jnp.bfloat16))

    out = _sc_combine(y, d0, d1, w0a, w1a)
    return out.reshape(bsz, seqlen, dim)
